# baseline (speedup 1.0000x reference)
import functools

import jax
import jax.numpy as jnp
from jax import lax
from jax.experimental import pallas as pl
from jax.experimental.pallas import tpu as pltpu
from jax.experimental.pallas import tpu_sc as plsc

B = 16384
D = 64
WIDTH = 128

_NC = 2
_NS = 16
_NW = _NC * _NS
_BPW = B // _NW
_CHUNK = 128
_NCHUNK = _BPW // _CHUNK


def _sc_gather(e1r, relr, e2r, ent2, rel2):
    mesh = plsc.VectorSubcoreMesh(core_axis_name="c", subcore_axis_name="s")

    @functools.partial(
        pl.kernel,
        mesh=mesh,
        out_type=(
            jax.ShapeDtypeStruct((B, 2 * D), jnp.float32),
            jax.ShapeDtypeStruct((B, 2 * D), jnp.float32),
            jax.ShapeDtypeStruct((B, 2 * D), jnp.float32),
        ),
        scratch_types=[
            pltpu.VMEM((_NCHUNK, _CHUNK), jnp.int32),
            pltpu.VMEM((_NCHUNK, _CHUNK), jnp.int32),
            pltpu.VMEM((_NCHUNK, _CHUNK), jnp.int32),
            pltpu.VMEM((_BPW, 2 * D), jnp.float32),
            pltpu.SemaphoreType.DMA,
        ],
        compiler_params=pltpu.CompilerParams(use_tc_tiling_on_sc=True),
    )
    def k(e1_hbm, rel_hbm, e2_hbm, ent_hbm, relemb_hbm,
          oh_hbm, or_hbm, ot_hbm,
          ih_v, ir_v, it_v, rows_v, sem):
        wid = lax.axis_index("s") * _NC + lax.axis_index("c")
        row0 = wid * _NCHUNK
        base = wid * _BPW
        pltpu.sync_copy(e1_hbm.at[pl.ds(row0, _NCHUNK)], ih_v)
        pltpu.sync_copy(rel_hbm.at[pl.ds(row0, _NCHUNK)], ir_v)
        pltpu.sync_copy(e2_hbm.at[pl.ds(row0, _NCHUNK)], it_v)
        for idx_v, table, out in ((ih_v, ent_hbm, oh_hbm),
                                  (ir_v, relemb_hbm, or_hbm),
                                  (it_v, ent_hbm, ot_hbm)):
            copies = []
            for j in range(_NCHUNK):
                copies.append(pltpu.async_copy(
                    table.at[idx_v.at[j]],
                    rows_v.at[pl.ds(j * _CHUNK, _CHUNK)], sem))
            for cp in copies:
                cp.wait()
            pltpu.sync_copy(rows_v, out.at[pl.ds(base, _BPW)])

    return k(e1r, relr, e2r, ent2, rel2)


_NB = 16
_BLK = B // _NB


def _mlp_body(h2_ref, r2_ref, t2_ref, ph_ref, pr_ref, pt_ref,
              w1h, w1r, w1t, b1, gamma, beta, w2, b2, out_ref, acc_ref):
    phase = pl.program_id(0)
    i = pl.program_id(1)
    h = jnp.where(ph_ref[...] > 0.5, h2_ref[:, D:], h2_ref[:, :D])
    r = jnp.where(pr_ref[...] > 0.5, r2_ref[:, D:], r2_ref[:, :D])
    t = jnp.where(pt_ref[...] > 0.5, t2_ref[:, D:], t2_ref[:, :D])
    y = jnp.dot(h, w1h[...], preferred_element_type=jnp.float32)
    y = y + jnp.dot(r, w1r[...], preferred_element_type=jnp.float32)
    y = y + jnp.dot(t, w1t[...], preferred_element_type=jnp.float32)
    y = y + b1[...]

    @pl.when(jnp.logical_and(phase == 0, i == 0))
    def _():
        acc_ref[...] = jnp.zeros_like(acc_ref)

    @pl.when(phase == 0)
    def _():
        acc_ref[0:1, :] += jnp.sum(y, axis=0, keepdims=True)
        acc_ref[1:2, :] += jnp.sum(y * y, axis=0, keepdims=True)

    @pl.when(phase == 1)
    def _():
        mean = acc_ref[0:1, :] * (1.0 / B)
        var = acc_ref[1:2, :] * (1.0 / B) - mean * mean
        z = (y - mean) * (gamma[...] * lax.rsqrt(var + 1e-5)) + beta[...]
        z = jnp.maximum(z, 0.0)
        o = jnp.dot(z, w2[...], preferred_element_type=jnp.float32) + b2[...]
        out_ref[...] = jax.nn.sigmoid(o)


def kernel(e1_idx, rel_idx, e2_idx, entity_emb, relation_emb,
           W1, b1, gamma, beta, W2, b2):
    e1_idx = e1_idx.astype(jnp.int32)
    rel_idx = rel_idx.astype(jnp.int32)
    e2_idx = e2_idx.astype(jnp.int32)
    ent2 = entity_emb.reshape(500000, 2 * D)
    rel2 = relation_emb.reshape(500, 2 * D)
    e1r = (e1_idx >> 1).reshape(B // _CHUNK, _CHUNK)
    relr = (rel_idx >> 1).reshape(B // _CHUNK, _CHUNK)
    e2r = (e2_idx >> 1).reshape(B // _CHUNK, _CHUNK)

    h2, r2, t2 = _sc_gather(e1r, relr, e2r, ent2, rel2)

    def par(i):
        return jnp.broadcast_to((i & 1).astype(jnp.float32).reshape(B, 1),
                                (B, D))

    blk = lambda p, i: (i, 0)
    whole = lambda p, i: (0, 0)
    out = pl.pallas_call(
        _mlp_body,
        grid=(2, _NB),
        in_specs=[
            pl.BlockSpec((_BLK, 2 * D), blk),
            pl.BlockSpec((_BLK, 2 * D), blk),
            pl.BlockSpec((_BLK, 2 * D), blk),
            pl.BlockSpec((_BLK, D), blk),
            pl.BlockSpec((_BLK, D), blk),
            pl.BlockSpec((_BLK, D), blk),
            pl.BlockSpec((D, WIDTH), whole),
            pl.BlockSpec((D, WIDTH), whole),
            pl.BlockSpec((D, WIDTH), whole),
            pl.BlockSpec((1, WIDTH), whole),
            pl.BlockSpec((1, WIDTH), whole),
            pl.BlockSpec((1, WIDTH), whole),
            pl.BlockSpec((WIDTH, 1), whole),
            pl.BlockSpec((1, 1), whole),
        ],
        out_specs=pl.BlockSpec((_BLK, 1), blk),
        scratch_shapes=[pltpu.VMEM((2, WIDTH), jnp.float32)],
        out_shape=jax.ShapeDtypeStruct((B, 1), jnp.float32),
    )(h2, r2, t2, par(e1_idx), par(rel_idx), par(e2_idx),
      W1[:D], W1[D:2 * D], W1[2 * D:],
      b1.reshape(1, WIDTH), gamma.reshape(1, WIDTH), beta.reshape(1, WIDTH),
      W2, b2.reshape(1, 1))
    return out
